# Initial kernel scaffold; baseline (speedup 1.0000x reference)
import jax
import jax.numpy as jnp
from jax.experimental import pallas as pl


def kernel(x, edge_index, W_in, b_in, W_c0, b_c0, W_c1, b_c1, g0, be0, g1, be1, W_out, b_out):
    raise NotImplementedError("write your pallas kernel here")


# speedup vs baseline: 6.2799x; 6.2799x over previous
import functools

import jax
import jax.numpy as jnp
from jax import lax
from jax.experimental import pallas as pl
from jax.experimental.pallas import tpu as pltpu
from jax.experimental.pallas import tpu_sc as plsc

N = 10000
E = 320000
D = 128

NC = 2
NS = 16
NW = NC * NS
EDGES_PER_W = E // NW
CHUNK = 80
NCHUNKS = EDGES_PER_W // CHUNK
N_PAD = 10240
ROWS_PER_TILE = N_PAD // NS
DEG_W = 128


def _sc_aggregate(h, row3, col3, zrows):
    mesh = plsc.VectorSubcoreMesh(core_axis_name="c", subcore_axis_name="s")

    out_type = jax.ShapeDtypeStruct((NC, N_PAD, D), jnp.float32)

    scratch = [
        pltpu.VMEM((NCHUNKS, CHUNK), jnp.int32),
        pltpu.VMEM((NCHUNKS, CHUNK), jnp.int32),
        pltpu.VMEM((CHUNK, D), jnp.float32),
        pltpu.VMEM_SHARED((N_PAD, D), jnp.float32),
        pltpu.SemaphoreType.DMA,
    ]

    def body(h_hbm, row_hbm, col_hbm, z_hbm, *rest):
        (acc_out, col_v, row_v, msgs_v, acc_sh, sem) = rest
        cid = lax.axis_index("c")
        sid = lax.axis_index("s")
        wid = sid * NC + cid

        pltpu.sync_copy(col_hbm.at[wid], col_v)
        pltpu.sync_copy(row_hbm.at[wid], row_v)

        base = sid * ROWS_PER_TILE
        pltpu.sync_copy(z_hbm, acc_sh.at[pl.ds(base, ROWS_PER_TILE)])
        plsc.subcore_barrier()

        def chunk_step(j, _):
            pltpu.async_copy(h_hbm.at[col_v.at[j]], msgs_v, sem).wait()
            pltpu.sync_copy(msgs_v, acc_sh.at[row_v.at[j]], add=True)
            return 0

        lax.fori_loop(0, NCHUNKS, chunk_step, 0)
        plsc.subcore_barrier()

        pltpu.sync_copy(acc_sh.at[pl.ds(base, ROWS_PER_TILE)],
                        acc_out.at[cid, pl.ds(base, ROWS_PER_TILE)])

    run = pl.kernel(body, out_type=out_type, mesh=mesh,
                    scratch_types=scratch)
    return run(h, row3, col3, zrows)


def _sc_degree(row3, ones, zrows):
    mesh = plsc.VectorSubcoreMesh(core_axis_name="c", subcore_axis_name="s")
    out_type = jax.ShapeDtypeStruct((NC, N_PAD, DEG_W), jnp.float32)
    scratch = [
        pltpu.VMEM((NCHUNKS, CHUNK), jnp.int32),
        pltpu.VMEM((CHUNK, DEG_W), jnp.float32),
        pltpu.VMEM_SHARED((N_PAD, DEG_W), jnp.float32),
    ]

    def body(row_hbm, ones_hbm, zd_hbm, deg_out, row_v, ones_v, deg_sh):
        cid = lax.axis_index("c")
        sid = lax.axis_index("s")
        wid = sid * NC + cid
        pltpu.sync_copy(row_hbm.at[wid], row_v)
        pltpu.sync_copy(ones_hbm, ones_v)
        base = sid * ROWS_PER_TILE
        pltpu.sync_copy(zd_hbm, deg_sh.at[pl.ds(base, ROWS_PER_TILE)])
        plsc.subcore_barrier()

        def chunk_step(j, _):
            pltpu.sync_copy(ones_v, deg_sh.at[row_v.at[j]], add=True)
            return 0

        lax.fori_loop(0, NCHUNKS, chunk_step, 0)
        plsc.subcore_barrier()
        pltpu.sync_copy(deg_sh.at[pl.ds(base, ROWS_PER_TILE)],
                        deg_out.at[cid, pl.ds(base, ROWS_PER_TILE)])

    run = pl.kernel(body, out_type=out_type, mesh=mesh,
                    scratch_types=scratch)
    return run(row3, ones, zrows)


BLK = 1000


def _inproj_body(x_ref, w_ref, b_ref, o_ref):
    o_ref[...] = (jnp.dot(x_ref[...], w_ref[...],
                          preferred_element_type=jnp.float32) + b_ref[...])


def _inproj(x, W, b):
    return pl.pallas_call(
        _inproj_body,
        grid=(N // BLK,),
        in_specs=[
            pl.BlockSpec((BLK, D), lambda i: (i, 0)),
            pl.BlockSpec((D, D), lambda i: (0, 0)),
            pl.BlockSpec((1, D), lambda i: (0, 0)),
        ],
        out_specs=pl.BlockSpec((BLK, D), lambda i: (i, 0)),
        out_shape=jax.ShapeDtypeStruct((N, D), jnp.float32),
    )(x, W, b.reshape(1, D))


def _combine_body(final, h_ref, acc_ref, deg_ref, wc_ref, bc_ref, g_ref,
                  be_ref, wo_ref, bo_ref, o_ref):
    agg = acc_ref[0] + acc_ref[1]
    deg = deg_ref[0, :, 0:1] + deg_ref[1, :, 0:1]
    agg = agg / jnp.maximum(deg, 1.0)
    hn = (jnp.dot(agg, wc_ref[...], preferred_element_type=jnp.float32)
          + bc_ref[...])
    h = h_ref[...] + hn
    mu = jnp.mean(h, axis=-1, keepdims=True)
    var = jnp.mean((h - mu) ** 2, axis=-1, keepdims=True)
    h = (h - mu) * lax.rsqrt(var + 1e-5) * g_ref[...] + be_ref[...]
    h = jnp.maximum(h, 0.0)
    if final:
        h = (jnp.dot(h, wo_ref[...], preferred_element_type=jnp.float32)
             + bo_ref[...])
    o_ref[...] = h


def _combine(h, acc, deg, Wc, bc, g, be, Wo, bo, final):
    return pl.pallas_call(
        functools.partial(_combine_body, final),
        grid=(N // BLK,),
        in_specs=[
            pl.BlockSpec((BLK, D), lambda i: (i, 0)),
            pl.BlockSpec((NC, BLK, D), lambda i: (0, i, 0)),
            pl.BlockSpec((NC, BLK, DEG_W), lambda i: (0, i, 0)),
            pl.BlockSpec((D, D), lambda i: (0, 0)),
            pl.BlockSpec((1, D), lambda i: (0, 0)),
            pl.BlockSpec((1, D), lambda i: (0, 0)),
            pl.BlockSpec((1, D), lambda i: (0, 0)),
            pl.BlockSpec((D, D), lambda i: (0, 0)),
            pl.BlockSpec((1, D), lambda i: (0, 0)),
        ],
        out_specs=pl.BlockSpec((BLK, D), lambda i: (i, 0)),
        out_shape=jax.ShapeDtypeStruct((N, D), jnp.float32),
    )(h, acc, deg, Wc, bc.reshape(1, D), g.reshape(1, D), be.reshape(1, D),
      Wo, bo.reshape(1, D))


def kernel(x, edge_index, W_in, b_in, W_c0, b_c0, W_c1, b_c1, g0, be0, g1,
           be1, W_out, b_out):
    row3 = edge_index[0].reshape(NW, NCHUNKS, CHUNK)
    col3 = edge_index[1].reshape(NW, NCHUNKS, CHUNK)
    zrows = jnp.zeros((ROWS_PER_TILE, D), jnp.float32)
    ones = jnp.ones((CHUNK, DEG_W), jnp.float32)

    h0 = _inproj(x, W_in, b_in)
    deg = _sc_degree(row3, ones, zrows)
    acc0 = _sc_aggregate(h0, row3, col3, zrows)
    h1 = _combine(h0, acc0, deg, W_c0, b_c0, g0, be0, W_out, b_out,
                  final=False)
    acc1 = _sc_aggregate(h1, row3, col3, zrows)
    out = _combine(h1, acc1, deg, W_c1, b_c1, g1, be1, W_out, b_out,
                   final=True)
    return out


# speedup vs baseline: 6.6354x; 1.0566x over previous
import functools

import jax
import jax.numpy as jnp
from jax import lax
from jax.experimental import pallas as pl
from jax.experimental.pallas import tpu as pltpu
from jax.experimental.pallas import tpu_sc as plsc

N = 10000
E = 320000
D = 128

NC = 2
NS = 16
NW = NC * NS
EDGES_PER_W = E // NW
CHUNK = 80
NCHUNKS = EDGES_PER_W // CHUNK
N_PAD = 10240
ROWS_PER_TILE = N_PAD // NS
DEG_W = 128


def _sc_aggregate(h, row3, col3, zrows):
    mesh = plsc.VectorSubcoreMesh(core_axis_name="c", subcore_axis_name="s")

    out_type = jax.ShapeDtypeStruct((NC, N_PAD, D), jnp.float32)

    scratch = [
        pltpu.VMEM((CHUNK,), jnp.int32),
        pltpu.VMEM((CHUNK,), jnp.int32),
        pltpu.VMEM((CHUNK,), jnp.int32),
        pltpu.VMEM((CHUNK,), jnp.int32),
        pltpu.VMEM((CHUNK, D), jnp.float32),
        pltpu.VMEM((CHUNK, D), jnp.float32),
        pltpu.VMEM_SHARED((N_PAD, D), jnp.float32),
        pltpu.SemaphoreType.DMA,
    ]

    def body(h_hbm, row_hbm, col_hbm, z_hbm, *rest):
        (acc_out, c0, c1, r0, r1, m0, m1, acc_sh, sem) = rest
        cid = lax.axis_index("c")
        sid = lax.axis_index("s")
        wid = sid * NC + cid

        base = sid * ROWS_PER_TILE
        pltpu.sync_copy(z_hbm, acc_sh.at[pl.ds(base, ROWS_PER_TILE)])
        plsc.subcore_barrier()

        pltpu.sync_copy(col_hbm.at[wid, 0], c0)
        pltpu.sync_copy(row_hbm.at[wid, 0], r0)
        pltpu.async_copy(h_hbm.at[c0], m0, sem)

        def pair_step(t, _):
            j0 = 2 * t
            j1 = j0 + 1
            pltpu.sync_copy(col_hbm.at[wid, j1], c1)
            pltpu.sync_copy(row_hbm.at[wid, j1], r1)
            pltpu.make_async_copy(h_hbm.at[c0], m0, sem).wait()
            pltpu.async_copy(h_hbm.at[c1], m1, sem)
            pltpu.sync_copy(m0, acc_sh.at[r0], add=True)
            pltpu.sync_copy(col_hbm.at[wid, j0 + 2], c0)
            pltpu.sync_copy(row_hbm.at[wid, j0 + 2], r0)
            pltpu.make_async_copy(h_hbm.at[c1], m1, sem).wait()
            pltpu.async_copy(h_hbm.at[c0], m0, sem)
            pltpu.sync_copy(m1, acc_sh.at[r1], add=True)
            return 0

        lax.fori_loop(0, (NCHUNKS - 1) // 2, pair_step, 0)
        pltpu.make_async_copy(h_hbm.at[c0], m0, sem).wait()
        pltpu.sync_copy(m0, acc_sh.at[r0], add=True)
        plsc.subcore_barrier()

        pltpu.sync_copy(acc_sh.at[pl.ds(base, ROWS_PER_TILE)],
                        acc_out.at[cid, pl.ds(base, ROWS_PER_TILE)])

    run = pl.kernel(body, out_type=out_type, mesh=mesh,
                    scratch_types=scratch)
    return run(h, row3, col3, zrows)


def _sc_degree(row3, ones, zrows):
    mesh = plsc.VectorSubcoreMesh(core_axis_name="c", subcore_axis_name="s")
    out_type = jax.ShapeDtypeStruct((NC, N_PAD, DEG_W), jnp.float32)
    scratch = [
        pltpu.VMEM((NCHUNKS, CHUNK), jnp.int32),
        pltpu.VMEM((CHUNK, DEG_W), jnp.float32),
        pltpu.VMEM_SHARED((N_PAD, DEG_W), jnp.float32),
    ]

    def body(row_hbm, ones_hbm, zd_hbm, deg_out, row_v, ones_v, deg_sh):
        cid = lax.axis_index("c")
        sid = lax.axis_index("s")
        wid = sid * NC + cid
        pltpu.sync_copy(row_hbm.at[wid], row_v)
        pltpu.sync_copy(ones_hbm, ones_v)
        base = sid * ROWS_PER_TILE
        pltpu.sync_copy(zd_hbm, deg_sh.at[pl.ds(base, ROWS_PER_TILE)])
        plsc.subcore_barrier()

        def chunk_step(j, _):
            pltpu.sync_copy(ones_v, deg_sh.at[row_v.at[j]], add=True)
            return 0

        lax.fori_loop(0, NCHUNKS, chunk_step, 0)
        plsc.subcore_barrier()
        pltpu.sync_copy(deg_sh.at[pl.ds(base, ROWS_PER_TILE)],
                        deg_out.at[cid, pl.ds(base, ROWS_PER_TILE)])

    run = pl.kernel(body, out_type=out_type, mesh=mesh,
                    scratch_types=scratch)
    return run(row3, ones, zrows)


BLK = 1000


def _inproj_body(x_ref, w_ref, b_ref, o_ref):
    o_ref[...] = (jnp.dot(x_ref[...], w_ref[...],
                          preferred_element_type=jnp.float32) + b_ref[...])


def _inproj(x, W, b):
    return pl.pallas_call(
        _inproj_body,
        grid=(N // BLK,),
        in_specs=[
            pl.BlockSpec((BLK, D), lambda i: (i, 0)),
            pl.BlockSpec((D, D), lambda i: (0, 0)),
            pl.BlockSpec((1, D), lambda i: (0, 0)),
        ],
        out_specs=pl.BlockSpec((BLK, D), lambda i: (i, 0)),
        out_shape=jax.ShapeDtypeStruct((N, D), jnp.float32),
    )(x, W, b.reshape(1, D))


def _combine_body(final, h_ref, acc_ref, deg_ref, wc_ref, bc_ref, g_ref,
                  be_ref, wo_ref, bo_ref, o_ref):
    agg = acc_ref[0] + acc_ref[1]
    deg = deg_ref[0, :, 0:1] + deg_ref[1, :, 0:1]
    agg = agg / jnp.maximum(deg, 1.0)
    hn = (jnp.dot(agg, wc_ref[...], preferred_element_type=jnp.float32)
          + bc_ref[...])
    h = h_ref[...] + hn
    mu = jnp.mean(h, axis=-1, keepdims=True)
    var = jnp.mean((h - mu) ** 2, axis=-1, keepdims=True)
    h = (h - mu) * lax.rsqrt(var + 1e-5) * g_ref[...] + be_ref[...]
    h = jnp.maximum(h, 0.0)
    if final:
        h = (jnp.dot(h, wo_ref[...], preferred_element_type=jnp.float32)
             + bo_ref[...])
    o_ref[...] = h


def _combine(h, acc, deg, Wc, bc, g, be, Wo, bo, final):
    return pl.pallas_call(
        functools.partial(_combine_body, final),
        grid=(N // BLK,),
        in_specs=[
            pl.BlockSpec((BLK, D), lambda i: (i, 0)),
            pl.BlockSpec((NC, BLK, D), lambda i: (0, i, 0)),
            pl.BlockSpec((NC, BLK, DEG_W), lambda i: (0, i, 0)),
            pl.BlockSpec((D, D), lambda i: (0, 0)),
            pl.BlockSpec((1, D), lambda i: (0, 0)),
            pl.BlockSpec((1, D), lambda i: (0, 0)),
            pl.BlockSpec((1, D), lambda i: (0, 0)),
            pl.BlockSpec((D, D), lambda i: (0, 0)),
            pl.BlockSpec((1, D), lambda i: (0, 0)),
        ],
        out_specs=pl.BlockSpec((BLK, D), lambda i: (i, 0)),
        out_shape=jax.ShapeDtypeStruct((N, D), jnp.float32),
    )(h, acc, deg, Wc, bc.reshape(1, D), g.reshape(1, D), be.reshape(1, D),
      Wo, bo.reshape(1, D))


def kernel(x, edge_index, W_in, b_in, W_c0, b_c0, W_c1, b_c1, g0, be0, g1,
           be1, W_out, b_out):
    row3 = edge_index[0].reshape(NW, NCHUNKS, CHUNK)
    col3 = edge_index[1].reshape(NW, NCHUNKS, CHUNK)
    zrows = jnp.zeros((ROWS_PER_TILE, D), jnp.float32)
    ones = jnp.ones((CHUNK, DEG_W), jnp.float32)

    h0 = _inproj(x, W_in, b_in)
    deg = _sc_degree(row3, ones, zrows)
    acc0 = _sc_aggregate(h0, row3, col3, zrows)
    h1 = _combine(h0, acc0, deg, W_c0, b_c0, g0, be0, W_out, b_out,
                  final=False)
    acc1 = _sc_aggregate(h1, row3, col3, zrows)
    out = _combine(h1, acc1, deg, W_c1, b_c1, g1, be1, W_out, b_out,
                   final=True)
    return out


# speedup vs baseline: 7.6864x; 1.1584x over previous
import functools

import jax
import jax.numpy as jnp
from jax import lax
from jax.experimental import pallas as pl
from jax.experimental.pallas import tpu as pltpu
from jax.experimental.pallas import tpu_sc as plsc

N = 10000
E = 320000
D = 128

NC = 2
NS = 16
NW = NC * NS
EDGES_PER_W = E // NW
CHUNK = 80
NCHUNKS = EDGES_PER_W // CHUNK
N_PAD = 10240
ROWS_PER_TILE = N_PAD // NS
DEG_W = 128


def _sc_aggregate(h, packed3, zrows):
    mesh = plsc.VectorSubcoreMesh(core_axis_name="c", subcore_axis_name="s")

    out_type = jax.ShapeDtypeStruct((NC, N_PAD, D), jnp.float32)

    scratch = [
        pltpu.VMEM((NCHUNKS, CHUNK), jnp.int32),
        pltpu.VMEM((CHUNK,), jnp.int32),
        pltpu.VMEM((CHUNK,), jnp.int32),
        pltpu.VMEM((CHUNK,), jnp.int32),
        pltpu.VMEM((CHUNK,), jnp.int32),
        pltpu.VMEM((CHUNK, D), jnp.float32),
        pltpu.VMEM((CHUNK, D), jnp.float32),
        pltpu.VMEM_SHARED((N_PAD, D), jnp.float32),
        pltpu.SemaphoreType.DMA,
    ]

    def body(h_hbm, packed_hbm, z_hbm, *rest):
        (acc_out, packed_v, c0, c1, r0, r1, m0, m1, acc_sh, sem) = rest
        cid = lax.axis_index("c")
        sid = lax.axis_index("s")
        wid = sid * NC + cid

        pltpu.sync_copy(packed_hbm.at[wid], packed_v)

        base = sid * ROWS_PER_TILE
        pltpu.sync_copy(z_hbm, acc_sh.at[pl.ds(base, ROWS_PER_TILE)])
        plsc.subcore_barrier()

        def unpack(j, c_buf, r_buf):
            for k in range(CHUNK // 16):
                p = packed_v[j, pl.ds(k * 16, 16)]
                c_buf[pl.ds(k * 16, 16)] = jnp.bitwise_and(p, 0xFFFF)
                r_buf[pl.ds(k * 16, 16)] = lax.shift_right_logical(p, 16)

        unpack(0, c0, r0)
        pltpu.async_copy(h_hbm.at[c0], m0, sem)

        def pair_step(t, _):
            j0 = 2 * t
            j1 = j0 + 1
            unpack(j1, c1, r1)
            pltpu.make_async_copy(h_hbm.at[c0], m0, sem).wait()
            pltpu.async_copy(h_hbm.at[c1], m1, sem)
            pltpu.sync_copy(m0, acc_sh.at[r0], add=True)
            unpack(j0 + 2, c0, r0)
            pltpu.make_async_copy(h_hbm.at[c1], m1, sem).wait()
            pltpu.async_copy(h_hbm.at[c0], m0, sem)
            pltpu.sync_copy(m1, acc_sh.at[r1], add=True)
            return 0

        lax.fori_loop(0, (NCHUNKS - 1) // 2, pair_step, 0)
        pltpu.make_async_copy(h_hbm.at[c0], m0, sem).wait()
        pltpu.sync_copy(m0, acc_sh.at[r0], add=True)
        plsc.subcore_barrier()

        pltpu.sync_copy(acc_sh.at[pl.ds(base, ROWS_PER_TILE)],
                        acc_out.at[cid, pl.ds(base, ROWS_PER_TILE)])

    run = pl.kernel(body, out_type=out_type, mesh=mesh,
                    scratch_types=scratch)
    return run(h, packed3, zrows)


def _sc_degree(row3, ones, zrows):
    mesh = plsc.VectorSubcoreMesh(core_axis_name="c", subcore_axis_name="s")
    out_type = jax.ShapeDtypeStruct((NC, N_PAD, DEG_W), jnp.float32)
    scratch = [
        pltpu.VMEM((NCHUNKS, CHUNK), jnp.int32),
        pltpu.VMEM((CHUNK, DEG_W), jnp.float32),
        pltpu.VMEM_SHARED((N_PAD, DEG_W), jnp.float32),
    ]

    def body(row_hbm, ones_hbm, zd_hbm, deg_out, row_v, ones_v, deg_sh):
        cid = lax.axis_index("c")
        sid = lax.axis_index("s")
        wid = sid * NC + cid
        pltpu.sync_copy(row_hbm.at[wid], row_v)
        pltpu.sync_copy(ones_hbm, ones_v)
        base = sid * ROWS_PER_TILE
        pltpu.sync_copy(zd_hbm, deg_sh.at[pl.ds(base, ROWS_PER_TILE)])
        plsc.subcore_barrier()

        def chunk_step(j, _):
            pltpu.sync_copy(ones_v, deg_sh.at[row_v.at[j]], add=True)
            return 0

        lax.fori_loop(0, NCHUNKS, chunk_step, 0)
        plsc.subcore_barrier()
        pltpu.sync_copy(deg_sh.at[pl.ds(base, ROWS_PER_TILE)],
                        deg_out.at[cid, pl.ds(base, ROWS_PER_TILE)])

    run = pl.kernel(body, out_type=out_type, mesh=mesh,
                    scratch_types=scratch)
    return run(row3, ones, zrows)


BLK = 1000


def _inproj_body(x_ref, w_ref, b_ref, o_ref):
    o_ref[...] = (jnp.dot(x_ref[...], w_ref[...],
                          preferred_element_type=jnp.float32) + b_ref[...])


def _inproj(x, W, b):
    return pl.pallas_call(
        _inproj_body,
        grid=(N // BLK,),
        in_specs=[
            pl.BlockSpec((BLK, D), lambda i: (i, 0)),
            pl.BlockSpec((D, D), lambda i: (0, 0)),
            pl.BlockSpec((1, D), lambda i: (0, 0)),
        ],
        out_specs=pl.BlockSpec((BLK, D), lambda i: (i, 0)),
        out_shape=jax.ShapeDtypeStruct((N, D), jnp.float32),
    )(x, W, b.reshape(1, D))


def _combine_body(final, h_ref, acc_ref, deg_ref, wc_ref, bc_ref, g_ref,
                  be_ref, wo_ref, bo_ref, o_ref):
    agg = acc_ref[0] + acc_ref[1]
    deg = deg_ref[0, :, 0:1] + deg_ref[1, :, 0:1]
    agg = agg / jnp.maximum(deg, 1.0)
    hn = (jnp.dot(agg, wc_ref[...], preferred_element_type=jnp.float32)
          + bc_ref[...])
    h = h_ref[...] + hn
    mu = jnp.mean(h, axis=-1, keepdims=True)
    var = jnp.mean((h - mu) ** 2, axis=-1, keepdims=True)
    h = (h - mu) * lax.rsqrt(var + 1e-5) * g_ref[...] + be_ref[...]
    h = jnp.maximum(h, 0.0)
    if final:
        h = (jnp.dot(h, wo_ref[...], preferred_element_type=jnp.float32)
             + bo_ref[...])
    o_ref[...] = h


def _combine(h, acc, deg, Wc, bc, g, be, Wo, bo, final):
    return pl.pallas_call(
        functools.partial(_combine_body, final),
        grid=(N // BLK,),
        in_specs=[
            pl.BlockSpec((BLK, D), lambda i: (i, 0)),
            pl.BlockSpec((NC, BLK, D), lambda i: (0, i, 0)),
            pl.BlockSpec((NC, BLK, DEG_W), lambda i: (0, i, 0)),
            pl.BlockSpec((D, D), lambda i: (0, 0)),
            pl.BlockSpec((1, D), lambda i: (0, 0)),
            pl.BlockSpec((1, D), lambda i: (0, 0)),
            pl.BlockSpec((1, D), lambda i: (0, 0)),
            pl.BlockSpec((D, D), lambda i: (0, 0)),
            pl.BlockSpec((1, D), lambda i: (0, 0)),
        ],
        out_specs=pl.BlockSpec((BLK, D), lambda i: (i, 0)),
        out_shape=jax.ShapeDtypeStruct((N, D), jnp.float32),
    )(h, acc, deg, Wc, bc.reshape(1, D), g.reshape(1, D), be.reshape(1, D),
      Wo, bo.reshape(1, D))


def kernel(x, edge_index, W_in, b_in, W_c0, b_c0, W_c1, b_c1, g0, be0, g1,
           be1, W_out, b_out):
    row3 = edge_index[0].reshape(NW, NCHUNKS, CHUNK)
    packed3 = ((edge_index[0] << 16) | edge_index[1]).reshape(
        NW, NCHUNKS, CHUNK)
    zrows = jnp.zeros((ROWS_PER_TILE, D), jnp.float32)
    ones = jnp.ones((CHUNK, DEG_W), jnp.float32)

    h0 = _inproj(x, W_in, b_in)
    deg = _sc_degree(row3, ones, zrows)
    acc0 = _sc_aggregate(h0, packed3, zrows)
    h1 = _combine(h0, acc0, deg, W_c0, b_c0, g0, be0, W_out, b_out,
                  final=False)
    acc1 = _sc_aggregate(h1, packed3, zrows)
    out = _combine(h1, acc1, deg, W_c1, b_c1, g1, be1, W_out, b_out,
                   final=True)
    return out
